# Initial kernel scaffold; baseline (speedup 1.0000x reference)
import jax
import jax.numpy as jnp
from jax.experimental import pallas as pl


def kernel(x, tables):
    raise NotImplementedError("write your pallas kernel here")


# speedup vs baseline: 1.0107x; 1.0107x over previous
import functools

import jax
import jax.numpy as jnp
from jax import lax
from jax.experimental import pallas as pl
from jax.experimental.pallas import tpu as pltpu
from jax.experimental.pallas import tpu_sc as plsc

NUM_FIELDS = 26
VOCAB = 100000
EMBED_DIM = 64
BATCH = 4096
LANES = 16


def _sc_body(tab_ref, xt_ref, out_ref, idx_v, buf_v, acc_v, sem):
    nc = 2
    wid = lax.axis_index("s") * nc + lax.axis_index("c")
    rows = BATCH // 32
    base = wid * rows

    for f in range(NUM_FIELDS):
        pltpu.sync_copy(xt_ref.at[f, pl.ds(base, rows)], idx_v)
        for i in range(rows // LANES):
            sl = pl.ds(i * LANES, LANES)
            idx_v[sl] = idx_v[sl] + f * VOCAB

        if f == 0:
            pltpu.async_copy(tab_ref.at[idx_v], acc_v, sem).wait()
        else:
            pltpu.async_copy(tab_ref.at[idx_v], buf_v, sem).wait()

            def accum(r, carry):
                for c in range(EMBED_DIM // LANES):
                    sl = pl.ds(c * LANES, LANES)
                    plsc.addupdate(acc_v.at[r, sl], buf_v[r, sl])
                return carry

            lax.fori_loop(0, rows, accum, 0)

    pltpu.sync_copy(acc_v, out_ref.at[pl.ds(base, rows)])


def kernel(x, tables):
    rows = BATCH // 32
    xt = x.astype(jnp.int32).T
    tab = tables.reshape(NUM_FIELDS * VOCAB, EMBED_DIM)
    mesh = plsc.VectorSubcoreMesh(core_axis_name="c", subcore_axis_name="s")
    run = functools.partial(
        pl.kernel,
        mesh=mesh,
        out_type=jax.ShapeDtypeStruct((BATCH, EMBED_DIM), jnp.float32),
        scratch_types=[
            pltpu.VMEM((rows,), jnp.int32),
            pltpu.VMEM((rows, EMBED_DIM), jnp.float32),
            pltpu.VMEM((rows, EMBED_DIM), jnp.float32),
            pltpu.SemaphoreType.DMA,
        ],
        compiler_params=pltpu.CompilerParams(use_tc_tiling_on_sc=False),
    )(_sc_body)
    return run(tab, xt)


# speedup vs baseline: 1.0337x; 1.0228x over previous
import functools

import jax
import jax.numpy as jnp
from jax import lax
from jax.experimental import pallas as pl
from jax.experimental.pallas import tpu as pltpu
from jax.experimental.pallas import tpu_sc as plsc

NUM_FIELDS = 26
VOCAB = 100000
EMBED_DIM = 64
BATCH = 4096
LANES = 16
NBUF = 4


def _sc_body(tab_ref, xt_ref, out_ref, idx_v, buf_v, acc_v, sems):
    nc = 2
    wid = lax.axis_index("s") * nc + lax.axis_index("c")
    rows = BATCH // 32
    base = wid * rows

    pltpu.sync_copy(xt_ref.at[:, pl.ds(base, rows)], idx_v)
    for f in range(1, NUM_FIELDS):
        for i in range(rows // LANES):
            sl = pl.ds(i * LANES, LANES)
            idx_v[f, sl] = idx_v[f, sl] + f * VOCAB

    descs = {}
    for f in range(NBUF):
        descs[f] = pltpu.async_copy(
            tab_ref.at[idx_v.at[f]], buf_v.at[f], sems.at[f])

    for f in range(NUM_FIELDS):
        slot = f % NBUF
        descs[f].wait()

        def accum(r, carry):
            for c in range(EMBED_DIM // LANES):
                sl = pl.ds(c * LANES, LANES)
                if f == 0:
                    acc_v[r, sl] = buf_v[slot, r, sl]
                else:
                    plsc.addupdate(acc_v.at[r, sl], buf_v[slot, r, sl])
            return carry

        lax.fori_loop(0, rows, accum, 0)

        nxt = f + NBUF
        if nxt < NUM_FIELDS:
            descs[nxt] = pltpu.async_copy(
                tab_ref.at[idx_v.at[nxt]], buf_v.at[slot], sems.at[slot])

    pltpu.sync_copy(acc_v, out_ref.at[pl.ds(base, rows)])


def kernel(x, tables):
    rows = BATCH // 32
    xt = x.astype(jnp.int32).T
    tab = tables.reshape(NUM_FIELDS * VOCAB, EMBED_DIM)
    mesh = plsc.VectorSubcoreMesh(core_axis_name="c", subcore_axis_name="s")
    run = functools.partial(
        pl.kernel,
        mesh=mesh,
        out_type=jax.ShapeDtypeStruct((BATCH, EMBED_DIM), jnp.float32),
        scratch_types=[
            pltpu.VMEM((NUM_FIELDS, rows), jnp.int32),
            pltpu.VMEM((NBUF, rows, EMBED_DIM), jnp.float32),
            pltpu.VMEM((rows, EMBED_DIM), jnp.float32),
            pltpu.SemaphoreType.DMA((NBUF,)),
        ],
        compiler_params=pltpu.CompilerParams(use_tc_tiling_on_sc=False),
    )(_sc_body)
    return run(tab, xt)
